# baseline (speedup 1.0000x reference)
import functools

import jax
import jax.numpy as jnp
from jax import lax
from jax.experimental import pallas as pl
from jax.experimental.pallas import tpu as pltpu
from jax.experimental.pallas import tpu_sc as plsc

_LANES = 16
_NC = 2
_NS = 16


def _encode_kernel(gtT_ref, extT_ref, tab_ref):
    gtT = gtT_ref[...]
    extT = extT_ref[...]
    h = gtT[6:7, :]
    nb = gtT.shape[1]
    tab_ref[...] = jnp.concatenate(
        [gtT[0:3, :],
         jnp.cos(h),
         jnp.sin(h),
         gtT[3:6, :] * 0.5,
         extT[3:6, :] * 0.5,
         jnp.log(jnp.maximum(gtT[3:6, :], 1e-3)),
         jnp.zeros((2, nb), jnp.float32)], axis=0)


def _bcast_lane(v, j):
    idx = jnp.full((_LANES,), j, jnp.int32)
    return jnp.take_along_axis(v, idx, axis=0, mode="promise_in_bounds")


def _assign_kernel(pts_ref, gtT_ref, extT_ref, cls_ref, box_ref, *,
                   n_boxes, m_per_b):
    pts = pts_ref[...]
    gtT = gtT_ref[...]
    extT = extT_ref[...]

    bs = pts[:, 0:1].astype(jnp.int32)
    x = pts[:, 1:2]
    y = pts[:, 2:3]
    z = pts[:, 3:4]

    np_ = pts.shape[0]
    lane = jax.lax.broadcasted_iota(jnp.int32, (np_, n_boxes), 1)
    bmask = (lane // m_per_b) == bs

    def in_flags(t):
        cosa = jnp.cos(t[6:7, :])
        sina = jnp.sin(t[6:7, :])
        sx = x - t[0:1, :]
        sy = y - t[1:2, :]
        sz = z - t[2:3, :]
        lx = sx * cosa + sy * sina
        ly = -sx * sina + sy * cosa
        return ((jnp.abs(lx) <= t[3:4, :] * 0.5)
                & (jnp.abs(ly) <= t[4:5, :] * 0.5)
                & (jnp.abs(sz) <= t[5:6, :] * 0.5)
                & bmask)

    inb = in_flags(gtT)
    ine = in_flags(extT)

    fg = jnp.any(inb, axis=1, keepdims=True)
    exta = jnp.any(ine, axis=1, keepdims=True)
    ignore = jnp.logical_xor(fg, exta)
    cls_ref[...] = jnp.where(ignore, -1, jnp.where(fg, 1, 0)).astype(jnp.int32)

    hit = jnp.min(jnp.where(inb, lane, n_boxes), axis=1, keepdims=True)
    hitb = lane == hit

    enc = jnp.concatenate(
        [gtT[0:3, :],
         jnp.log(jnp.maximum(gtT[3:6, :], 1e-3)),
         jnp.cos(gtT[6:7, :]),
         jnp.sin(gtT[6:7, :])], axis=0)

    g = jnp.concatenate(
        [jnp.sum(jnp.where(hitb, enc[r:r + 1, :], 0.0), axis=1, keepdims=True)
         for r in range(8)], axis=1)
    offs = g[:, 0:3] - jnp.concatenate([x, y, z], axis=1)
    box = jnp.concatenate([offs, g[:, 3:8]], axis=1)
    box_ref[...] = box * fg.astype(jnp.float32)


def _sc_body(tab_hbm, pts_hbm, cls_hbm, box_hbm, tab_v, pts_v, cls_v, box_v,
             *, g_lo, rem, nb, m_per_b, n_batches):
    wid = lax.axis_index("s") * _NC + lax.axis_index("c")
    main_pts = g_lo * _LANES
    has_tail = wid < rem
    base = wid * main_pts + jnp.minimum(wid, rem) * _LANES
    n_groups = jnp.where(has_tail, g_lo + 1, g_lo)
    pltpu.sync_copy(tab_hbm, tab_v)
    pltpu.sync_copy(pts_hbm.at[pl.ds(base * 4, main_pts * 4)],
                    pts_v.at[pl.ds(0, main_pts * 4)])

    @pl.when(has_tail)
    def _():
        pltpu.sync_copy(
            pts_hbm.at[pl.ds((base + main_pts) * 4, _LANES * 4)],
            pts_v.at[pl.ds(main_pts * 4, _LANES * 4)])

    big = jnp.int32(16384)

    nk = m_per_b // _LANES

    def group(g, carry):
        s = g * _LANES
        iota = lax.broadcasted_iota(jnp.int32, (_LANES,), 0)
        p4 = s * 4 + iota * 4
        bsv = plsc.load_gather(pts_v, [p4]).astype(jnp.int32)
        xv = plsc.load_gather(pts_v, [p4 + 1])
        yv = plsc.load_gather(pts_v, [p4 + 2])
        zv = plsc.load_gather(pts_v, [p4 + 3])

        def chunk_body(cc, st2):
            fh, ea = st2
            off = cc * _LANES
            pm = bsv == cc // nk
            cxv = tab_v[pl.ds(off, _LANES)]
            cyv = tab_v[pl.ds(nb + off, _LANES)]
            czv = tab_v[pl.ds(2 * nb + off, _LANES)]
            cav = tab_v[pl.ds(3 * nb + off, _LANES)]
            sav = tab_v[pl.ds(4 * nb + off, _LANES)]
            hxv = tab_v[pl.ds(5 * nb + off, _LANES)]
            hyv = tab_v[pl.ds(6 * nb + off, _LANES)]
            hzv = tab_v[pl.ds(7 * nb + off, _LANES)]
            exv = tab_v[pl.ds(8 * nb + off, _LANES)]
            eyv = tab_v[pl.ds(9 * nb + off, _LANES)]
            ezv = tab_v[pl.ds(10 * nb + off, _LANES)]

            def jbody(j, st3):
                fh, ea = st3
                sx = xv - _bcast_lane(cxv, j)
                sy = yv - _bcast_lane(cyv, j)
                sz = zv - _bcast_lane(czv, j)
                ca = _bcast_lane(cav, j)
                sa = _bcast_lane(sav, j)
                lx = sx * ca + sy * sa
                ly = -sx * sa + sy * ca
                alx = jnp.abs(lx)
                aly = jnp.abs(ly)
                alz = jnp.abs(sz)
                ing = ((alx <= _bcast_lane(hxv, j))
                       & (aly <= _bcast_lane(hyv, j))
                       & (alz <= _bcast_lane(hzv, j)) & pm)
                ine = ((alx <= _bcast_lane(exv, j))
                       & (aly <= _bcast_lane(eyv, j))
                       & (alz <= _bcast_lane(ezv, j)) & pm)
                fh = jnp.minimum(fh, jnp.where(ing, off + j, big))
                ea = jnp.where(ine, jnp.int32(1), ea)
                return fh, ea

            return lax.fori_loop(0, _LANES, jbody, (fh, ea))

        fh0 = jnp.full((_LANES,), big, jnp.int32)
        ea0 = jnp.zeros((_LANES,), jnp.int32)
        bmin = bsv[0]
        bmax = bsv[_LANES - 1]
        fh, ea = lax.fori_loop(bmin * nk, (bmax + 1) * nk, chunk_body,
                               (fh0, ea0))

        found = fh < big
        ign = jnp.logical_xor(found, ea != 0)
        cls = jnp.where(ign, -1, jnp.where(found, 1, 0)).astype(jnp.int32)
        cls_v[pl.ds(s, _LANES)] = cls

        safe = jnp.where(found, fh, 0)
        pvv = (xv, yv, zv)
        row_sel = (0, 1, 2, 11, 12, 13, 3, 4)
        for r_out in range(8):
            val = plsc.load_gather(tab_v, [row_sel[r_out] * nb + safe])
            if r_out < 3:
                val = val - pvv[r_out]
            val = jnp.where(found, val, 0.0)
            plsc.store_scatter(box_v, [s + iota, jnp.full((_LANES,), r_out,
                                                          jnp.int32)], val)
        return carry

    lax.fori_loop(0, n_groups, group, 0)
    pltpu.sync_copy(cls_v.at[pl.ds(0, main_pts)],
                    cls_hbm.at[pl.ds(base, main_pts)])
    pltpu.sync_copy(box_v.at[pl.ds(0, main_pts)],
                    box_hbm.at[pl.ds(base, main_pts)])

    @pl.when(has_tail)
    def _():
        pltpu.sync_copy(cls_v.at[pl.ds(main_pts, _LANES)],
                        cls_hbm.at[pl.ds(base + main_pts, _LANES)])
        pltpu.sync_copy(box_v.at[pl.ds(main_pts, _LANES)],
                        box_hbm.at[pl.ds(base + main_pts, _LANES)])


def kernel(points, gt_boxes, extend_gt_boxes):
    n = points.shape[0]
    b, m, c = gt_boxes.shape
    nb = b * m
    n_rows = 16
    gtT = gt_boxes.reshape(nb, c).T
    extT = extend_gt_boxes.reshape(nb, c).T

    table = pl.pallas_call(
        _encode_kernel,
        in_specs=[pl.BlockSpec((c, nb), lambda: (0, 0)),
                  pl.BlockSpec((c, nb), lambda: (0, 0))],
        out_specs=pl.BlockSpec((n_rows, nb), lambda: (0, 0)),
        out_shape=jax.ShapeDtypeStruct((n_rows, nb), jnp.float32),
    )(gtT, extT)

    nw = _NC * _NS
    g_total = n // _LANES
    g_lo = g_total // nw
    rem = g_total % nw
    tile_max = (g_lo + 1) * _LANES

    mesh = plsc.VectorSubcoreMesh(core_axis_name="c", subcore_axis_name="s")
    body = functools.partial(_sc_body, g_lo=g_lo, rem=rem,
                             nb=nb, m_per_b=m, n_batches=b)
    cls, box = pl.kernel(
        body,
        out_type=[jax.ShapeDtypeStruct((n,), jnp.int32),
                  jax.ShapeDtypeStruct((n, 8), jnp.float32)],
        mesh=mesh,
        compiler_params=pltpu.CompilerParams(needs_layout_passes=False),
        scratch_types=[
            pltpu.VMEM((n_rows * nb,), jnp.float32),
            pltpu.VMEM((tile_max * 4,), jnp.float32),
            pltpu.VMEM((tile_max,), jnp.int32),
            pltpu.VMEM((tile_max, 8), jnp.float32),
        ],
    )(table.reshape(-1), points.reshape(-1))

    return cls, box


# speedup vs baseline: 1.2193x; 1.2193x over previous
import functools

import jax
import jax.numpy as jnp
from jax import lax
from jax.experimental import pallas as pl
from jax.experimental.pallas import tpu as pltpu
from jax.experimental.pallas import tpu_sc as plsc

_LANES = 16
_NC = 2
_NS = 16


def _encode_kernel(gtT_ref, extT_ref, tab_ref):
    gtT = gtT_ref[...]
    extT = extT_ref[...]
    h = gtT[6:7, :]
    nb = gtT.shape[1]
    tab_ref[...] = jnp.concatenate(
        [gtT[0:3, :],
         jnp.cos(h),
         jnp.sin(h),
         gtT[3:6, :] * 0.5,
         extT[3:6, :] * 0.5,
         jnp.log(jnp.maximum(gtT[3:6, :], 1e-3)),
         jnp.zeros((2, nb), jnp.float32)], axis=0)


def _bcast_lane(v, j):
    idx = jnp.full((_LANES,), j, jnp.int32)
    return jnp.take_along_axis(v, idx, axis=0, mode="promise_in_bounds")


def _assign_kernel(pts_ref, gtT_ref, extT_ref, cls_ref, box_ref, *,
                   n_boxes, m_per_b):
    pts = pts_ref[...]
    gtT = gtT_ref[...]
    extT = extT_ref[...]

    bs = pts[:, 0:1].astype(jnp.int32)
    x = pts[:, 1:2]
    y = pts[:, 2:3]
    z = pts[:, 3:4]

    np_ = pts.shape[0]
    lane = jax.lax.broadcasted_iota(jnp.int32, (np_, n_boxes), 1)
    bmask = (lane // m_per_b) == bs

    def in_flags(t):
        cosa = jnp.cos(t[6:7, :])
        sina = jnp.sin(t[6:7, :])
        sx = x - t[0:1, :]
        sy = y - t[1:2, :]
        sz = z - t[2:3, :]
        lx = sx * cosa + sy * sina
        ly = -sx * sina + sy * cosa
        return ((jnp.abs(lx) <= t[3:4, :] * 0.5)
                & (jnp.abs(ly) <= t[4:5, :] * 0.5)
                & (jnp.abs(sz) <= t[5:6, :] * 0.5)
                & bmask)

    inb = in_flags(gtT)
    ine = in_flags(extT)

    fg = jnp.any(inb, axis=1, keepdims=True)
    exta = jnp.any(ine, axis=1, keepdims=True)
    ignore = jnp.logical_xor(fg, exta)
    cls_ref[...] = jnp.where(ignore, -1, jnp.where(fg, 1, 0)).astype(jnp.int32)

    hit = jnp.min(jnp.where(inb, lane, n_boxes), axis=1, keepdims=True)
    hitb = lane == hit

    enc = jnp.concatenate(
        [gtT[0:3, :],
         jnp.log(jnp.maximum(gtT[3:6, :], 1e-3)),
         jnp.cos(gtT[6:7, :]),
         jnp.sin(gtT[6:7, :])], axis=0)

    g = jnp.concatenate(
        [jnp.sum(jnp.where(hitb, enc[r:r + 1, :], 0.0), axis=1, keepdims=True)
         for r in range(8)], axis=1)
    offs = g[:, 0:3] - jnp.concatenate([x, y, z], axis=1)
    box = jnp.concatenate([offs, g[:, 3:8]], axis=1)
    box_ref[...] = box * fg.astype(jnp.float32)


def _sc_body(tab_hbm, pts_hbm, cls_hbm, box_hbm, tab_v, pts_v, cls_v, box_v,
             *, g_lo, rem, nb, m_per_b, n_batches, n_total, tile_max):
    wid = lax.axis_index("s") * _NC + lax.axis_index("c")
    main_pts = g_lo * _LANES
    has_tail = wid < rem
    base = wid * main_pts + jnp.minimum(wid, rem) * _LANES
    n_groups = jnp.where(has_tail, g_lo + 1, g_lo)
    pltpu.sync_copy(tab_hbm, tab_v)
    for r in range(4):
        pltpu.sync_copy(pts_hbm.at[pl.ds(r * n_total + base, main_pts)],
                        pts_v.at[pl.ds(r * tile_max, main_pts)])

    @pl.when(has_tail)
    def _():
        for r in range(4):
            pltpu.sync_copy(
                pts_hbm.at[pl.ds(r * n_total + base + main_pts, _LANES)],
                pts_v.at[pl.ds(r * tile_max + main_pts, _LANES)])

    big = jnp.int32(16384)

    nk = m_per_b // _LANES

    def group(g, carry):
        s = g * _LANES
        iota = lax.broadcasted_iota(jnp.int32, (_LANES,), 0)
        bsv = pts_v[pl.ds(s, _LANES)].astype(jnp.int32)
        xv = pts_v[pl.ds(tile_max + s, _LANES)]
        yv = pts_v[pl.ds(2 * tile_max + s, _LANES)]
        zv = pts_v[pl.ds(3 * tile_max + s, _LANES)]

        def chunk_body(cc, st2):
            fh, ea = st2
            off = cc * _LANES
            pm = bsv == cc // nk
            cxv = tab_v[0, pl.ds(off, _LANES)]
            cyv = tab_v[1, pl.ds(off, _LANES)]
            czv = tab_v[2, pl.ds(off, _LANES)]
            cav = tab_v[3, pl.ds(off, _LANES)]
            sav = tab_v[4, pl.ds(off, _LANES)]
            hxv = tab_v[5, pl.ds(off, _LANES)]
            hyv = tab_v[6, pl.ds(off, _LANES)]
            hzv = tab_v[7, pl.ds(off, _LANES)]
            exv = tab_v[8, pl.ds(off, _LANES)]
            eyv = tab_v[9, pl.ds(off, _LANES)]
            ezv = tab_v[10, pl.ds(off, _LANES)]

            def jbody(j, st3):
                fh, ea = st3
                sx = xv - _bcast_lane(cxv, j)
                sy = yv - _bcast_lane(cyv, j)
                sz = zv - _bcast_lane(czv, j)
                ca = _bcast_lane(cav, j)
                sa = _bcast_lane(sav, j)
                lx = sx * ca + sy * sa
                ly = -sx * sa + sy * ca
                alx = jnp.abs(lx)
                aly = jnp.abs(ly)
                alz = jnp.abs(sz)
                ing = ((alx <= _bcast_lane(hxv, j))
                       & (aly <= _bcast_lane(hyv, j))
                       & (alz <= _bcast_lane(hzv, j)) & pm)
                ine = ((alx <= _bcast_lane(exv, j))
                       & (aly <= _bcast_lane(eyv, j))
                       & (alz <= _bcast_lane(ezv, j)) & pm)
                fh = jnp.minimum(fh, jnp.where(ing, off + j, big))
                ea = jnp.where(ine, jnp.int32(1), ea)
                return fh, ea

            return lax.fori_loop(0, _LANES, jbody, (fh, ea))

        fh0 = jnp.full((_LANES,), big, jnp.int32)
        ea0 = jnp.zeros((_LANES,), jnp.int32)
        bmin = bsv[0]
        bmax = bsv[_LANES - 1]
        fh, ea = lax.fori_loop(bmin * nk, (bmax + 1) * nk, chunk_body,
                               (fh0, ea0))

        found = fh < big
        ign = jnp.logical_xor(found, ea != 0)
        cls = jnp.where(ign, -1, jnp.where(found, 1, 0)).astype(jnp.int32)
        cls_v[pl.ds(s, _LANES)] = cls

        safe = jnp.where(found, fh, 0)
        pvv = (xv, yv, zv)
        row_sel = (0, 1, 2, 11, 12, 13, 3, 4)
        for r_out in range(8):
            val = plsc.load_gather(tab_v, [jnp.full((_LANES,), row_sel[r_out], jnp.int32), safe])
            if r_out < 3:
                val = val - pvv[r_out]
            val = jnp.where(found, val, 0.0)
            plsc.store_scatter(box_v, [s + iota, jnp.full((_LANES,), r_out,
                                                          jnp.int32)], val)
        return carry

    lax.fori_loop(0, n_groups, group, 0)
    pltpu.sync_copy(cls_v.at[pl.ds(0, main_pts)],
                    cls_hbm.at[pl.ds(base, main_pts)])
    pltpu.sync_copy(box_v.at[pl.ds(0, main_pts)],
                    box_hbm.at[pl.ds(base, main_pts)])

    @pl.when(has_tail)
    def _():
        pltpu.sync_copy(cls_v.at[pl.ds(main_pts, _LANES)],
                        cls_hbm.at[pl.ds(base + main_pts, _LANES)])
        pltpu.sync_copy(box_v.at[pl.ds(main_pts, _LANES)],
                        box_hbm.at[pl.ds(base + main_pts, _LANES)])


def kernel(points, gt_boxes, extend_gt_boxes):
    n = points.shape[0]
    b, m, c = gt_boxes.shape
    nb = b * m
    n_rows = 16
    gtT = gt_boxes.reshape(nb, c).T
    extT = extend_gt_boxes.reshape(nb, c).T

    table = pl.pallas_call(
        _encode_kernel,
        in_specs=[pl.BlockSpec((c, nb), lambda: (0, 0)),
                  pl.BlockSpec((c, nb), lambda: (0, 0))],
        out_specs=pl.BlockSpec((n_rows, nb), lambda: (0, 0)),
        out_shape=jax.ShapeDtypeStruct((n_rows, nb), jnp.float32),
    )(gtT, extT)

    nw = _NC * _NS
    g_total = n // _LANES
    g_lo = g_total // nw
    rem = g_total % nw
    tile_max = (g_lo + 1) * _LANES

    mesh = plsc.VectorSubcoreMesh(core_axis_name="c", subcore_axis_name="s")
    body = functools.partial(_sc_body, g_lo=g_lo, rem=rem,
                             nb=nb, m_per_b=m, n_batches=b,
                             n_total=n, tile_max=tile_max)
    cls, box = pl.kernel(
        body,
        out_type=[jax.ShapeDtypeStruct((n,), jnp.int32),
                  jax.ShapeDtypeStruct((n, 8), jnp.float32)],
        mesh=mesh,
        compiler_params=pltpu.CompilerParams(needs_layout_passes=False),
        scratch_types=[
            pltpu.VMEM((n_rows, nb), jnp.float32),
            pltpu.VMEM((tile_max * 4,), jnp.float32),
            pltpu.VMEM((tile_max,), jnp.int32),
            pltpu.VMEM((tile_max, 8), jnp.float32),
        ],
    )(table, points.T.reshape(-1))

    return cls, box


# speedup vs baseline: 1.3136x; 1.0773x over previous
import functools

import jax
import jax.numpy as jnp
from jax import lax
from jax.experimental import pallas as pl
from jax.experimental.pallas import tpu as pltpu
from jax.experimental.pallas import tpu_sc as plsc

_LANES = 16
_NC = 2
_NS = 16


def _encode_kernel(gtT_ref, extT_ref, tab_ref):
    gtT = gtT_ref[...]
    extT = extT_ref[...]
    h = gtT[6:7, :]
    nb = gtT.shape[1]
    tab_ref[...] = jnp.concatenate(
        [gtT[0:3, :],
         jnp.cos(h),
         jnp.sin(h),
         gtT[3:6, :] * 0.5,
         extT[3:6, :] * 0.5,
         jnp.log(jnp.maximum(gtT[3:6, :], 1e-3)),
         jnp.zeros((2, nb), jnp.float32)], axis=0)


def _bcast_lane(v, j):
    idx = jnp.full((_LANES,), j, jnp.int32)
    return jnp.take_along_axis(v, idx, axis=0, mode="promise_in_bounds")


def _assign_kernel(pts_ref, gtT_ref, extT_ref, cls_ref, box_ref, *,
                   n_boxes, m_per_b):
    pts = pts_ref[...]
    gtT = gtT_ref[...]
    extT = extT_ref[...]

    bs = pts[:, 0:1].astype(jnp.int32)
    x = pts[:, 1:2]
    y = pts[:, 2:3]
    z = pts[:, 3:4]

    np_ = pts.shape[0]
    lane = jax.lax.broadcasted_iota(jnp.int32, (np_, n_boxes), 1)
    bmask = (lane // m_per_b) == bs

    def in_flags(t):
        cosa = jnp.cos(t[6:7, :])
        sina = jnp.sin(t[6:7, :])
        sx = x - t[0:1, :]
        sy = y - t[1:2, :]
        sz = z - t[2:3, :]
        lx = sx * cosa + sy * sina
        ly = -sx * sina + sy * cosa
        return ((jnp.abs(lx) <= t[3:4, :] * 0.5)
                & (jnp.abs(ly) <= t[4:5, :] * 0.5)
                & (jnp.abs(sz) <= t[5:6, :] * 0.5)
                & bmask)

    inb = in_flags(gtT)
    ine = in_flags(extT)

    fg = jnp.any(inb, axis=1, keepdims=True)
    exta = jnp.any(ine, axis=1, keepdims=True)
    ignore = jnp.logical_xor(fg, exta)
    cls_ref[...] = jnp.where(ignore, -1, jnp.where(fg, 1, 0)).astype(jnp.int32)

    hit = jnp.min(jnp.where(inb, lane, n_boxes), axis=1, keepdims=True)
    hitb = lane == hit

    enc = jnp.concatenate(
        [gtT[0:3, :],
         jnp.log(jnp.maximum(gtT[3:6, :], 1e-3)),
         jnp.cos(gtT[6:7, :]),
         jnp.sin(gtT[6:7, :])], axis=0)

    g = jnp.concatenate(
        [jnp.sum(jnp.where(hitb, enc[r:r + 1, :], 0.0), axis=1, keepdims=True)
         for r in range(8)], axis=1)
    offs = g[:, 0:3] - jnp.concatenate([x, y, z], axis=1)
    box = jnp.concatenate([offs, g[:, 3:8]], axis=1)
    box_ref[...] = box * fg.astype(jnp.float32)


def _sc_body(tab_hbm, pts_hbm, cls_hbm, box_hbm, tab_v, pts_v, cls_v, box_v,
             *, g_lo, rem, nb, m_per_b, n_batches, n_total, tile_max):
    wid = lax.axis_index("s") * _NC + lax.axis_index("c")
    main_pts = g_lo * _LANES
    has_tail = wid < rem
    base = wid * main_pts + jnp.minimum(wid, rem) * _LANES
    n_groups = jnp.where(has_tail, g_lo + 1, g_lo)
    pltpu.sync_copy(tab_hbm, tab_v)
    for r in range(4):
        pltpu.sync_copy(pts_hbm.at[pl.ds(r * n_total + base, main_pts)],
                        pts_v.at[pl.ds(r * tile_max, main_pts)])

    @pl.when(has_tail)
    def _():
        for r in range(4):
            pltpu.sync_copy(
                pts_hbm.at[pl.ds(r * n_total + base + main_pts, _LANES)],
                pts_v.at[pl.ds(r * tile_max + main_pts, _LANES)])

    big = jnp.int32(16384)

    nk = m_per_b // _LANES

    def group(g, carry):
        s = g * _LANES
        iota = lax.broadcasted_iota(jnp.int32, (_LANES,), 0)
        bsv = pts_v[pl.ds(s, _LANES)].astype(jnp.int32)
        xv = pts_v[pl.ds(tile_max + s, _LANES)]
        yv = pts_v[pl.ds(2 * tile_max + s, _LANES)]
        zv = pts_v[pl.ds(3 * tile_max + s, _LANES)]

        def chunk_body(cc, st2):
            fh, ea = st2
            off = cc * _LANES
            pm = bsv == cc // nk
            cxv = tab_v[0, pl.ds(off, _LANES)]
            cyv = tab_v[1, pl.ds(off, _LANES)]
            czv = tab_v[2, pl.ds(off, _LANES)]
            cav = tab_v[3, pl.ds(off, _LANES)]
            sav = tab_v[4, pl.ds(off, _LANES)]
            hxv = tab_v[5, pl.ds(off, _LANES)]
            hyv = tab_v[6, pl.ds(off, _LANES)]
            hzv = tab_v[7, pl.ds(off, _LANES)]
            exv = tab_v[8, pl.ds(off, _LANES)]
            eyv = tab_v[9, pl.ds(off, _LANES)]
            ezv = tab_v[10, pl.ds(off, _LANES)]

            def jbody(j, st3):
                fh, ea = st3
                sx = xv - _bcast_lane(cxv, j)
                sy = yv - _bcast_lane(cyv, j)
                sz = zv - _bcast_lane(czv, j)
                ca = _bcast_lane(cav, j)
                sa = _bcast_lane(sav, j)
                lx = sx * ca + sy * sa
                ly = -sx * sa + sy * ca
                alx = jnp.abs(lx)
                aly = jnp.abs(ly)
                alz = jnp.abs(sz)
                ing = ((alx <= _bcast_lane(hxv, j))
                       & (aly <= _bcast_lane(hyv, j))
                       & (alz <= _bcast_lane(hzv, j)) & pm)
                ine = ((alx <= _bcast_lane(exv, j))
                       & (aly <= _bcast_lane(eyv, j))
                       & (alz <= _bcast_lane(ezv, j)) & pm)
                fh = jnp.minimum(fh, jnp.where(ing, off + j, big))
                ea = jnp.where(ine, jnp.int32(1), ea)
                return fh, ea

            return lax.fori_loop(0, _LANES, jbody, (fh, ea))

        fh0 = jnp.full((_LANES,), big, jnp.int32)
        ea0 = jnp.zeros((_LANES,), jnp.int32)
        bmin = bsv[0]
        bmax = bsv[_LANES - 1]
        fh, ea = lax.fori_loop(bmin * nk, (bmax + 1) * nk, chunk_body,
                               (fh0, ea0))

        found = fh < big
        ign = jnp.logical_xor(found, ea != 0)
        cls = jnp.where(ign, -1, jnp.where(found, 1, 0)).astype(jnp.int32)
        cls_v[pl.ds(s, _LANES)] = cls

        safe = jnp.where(found, fh, 0)
        pvv = (xv, yv, zv)
        row_sel = (0, 1, 2, 11, 12, 13, 3, 4)
        for r_out in range(8):
            val = plsc.load_gather(tab_v, [jnp.full((_LANES,), row_sel[r_out], jnp.int32), safe])
            if r_out < 3:
                val = val - pvv[r_out]
            val = jnp.where(found, val, 0.0)
            plsc.store_scatter(box_v, [s + iota, jnp.full((_LANES,), r_out,
                                                          jnp.int32)], val)
        return carry

    lax.fori_loop(0, n_groups, group, 0)
    pltpu.sync_copy(cls_v.at[pl.ds(0, main_pts)],
                    cls_hbm.at[pl.ds(base, main_pts)])
    pltpu.sync_copy(box_v.at[pl.ds(0, main_pts)],
                    box_hbm.at[pl.ds(base, main_pts)])

    @pl.when(has_tail)
    def _():
        pltpu.sync_copy(cls_v.at[pl.ds(main_pts, _LANES)],
                        cls_hbm.at[pl.ds(base + main_pts, _LANES)])
        pltpu.sync_copy(box_v.at[pl.ds(main_pts, _LANES)],
                        box_hbm.at[pl.ds(base + main_pts, _LANES)])


def kernel(points, gt_boxes, extend_gt_boxes):
    n = points.shape[0]
    b, m, c = gt_boxes.shape
    nb = b * m
    n_rows = 16
    gtT = gt_boxes.reshape(nb, c).T
    extT = extend_gt_boxes.reshape(nb, c).T

    table = pl.pallas_call(
        _encode_kernel,
        in_specs=[pl.BlockSpec((c, nb), lambda: (0, 0)),
                  pl.BlockSpec((c, nb), lambda: (0, 0))],
        out_specs=pl.BlockSpec((n_rows, nb), lambda: (0, 0)),
        out_shape=jax.ShapeDtypeStruct((n_rows, nb), jnp.float32),
    )(gtT, extT)

    blk = 2000
    n_tc = min(n, 3 * blk)
    n_sc = n - n_tc

    nw = _NC * _NS
    g_total = n_sc // _LANES
    g_lo = g_total // nw
    rem = g_total % nw
    tile_max = (g_lo + 1) * _LANES

    mesh = plsc.VectorSubcoreMesh(core_axis_name="c", subcore_axis_name="s")
    body = functools.partial(_sc_body, g_lo=g_lo, rem=rem,
                             nb=nb, m_per_b=m, n_batches=b,
                             n_total=n_sc, tile_max=tile_max)
    cls_s, box_s = pl.kernel(
        body,
        out_type=[jax.ShapeDtypeStruct((n_sc,), jnp.int32),
                  jax.ShapeDtypeStruct((n_sc, 8), jnp.float32)],
        mesh=mesh,
        compiler_params=pltpu.CompilerParams(needs_layout_passes=False),
        scratch_types=[
            pltpu.VMEM((n_rows, nb), jnp.float32),
            pltpu.VMEM((tile_max * 4,), jnp.float32),
            pltpu.VMEM((tile_max,), jnp.int32),
            pltpu.VMEM((tile_max, 8), jnp.float32),
        ],
    )(table, points[n_tc:].T.reshape(-1))

    tc_body = functools.partial(_assign_kernel, n_boxes=nb, m_per_b=m)
    cls_t, box_t = pl.pallas_call(
        tc_body,
        grid=(n_tc // blk,),
        in_specs=[
            pl.BlockSpec((blk, 4), lambda i: (i, 0)),
            pl.BlockSpec((c, nb), lambda i: (0, 0)),
            pl.BlockSpec((c, nb), lambda i: (0, 0)),
        ],
        out_specs=[
            pl.BlockSpec((blk, 1), lambda i: (i, 0)),
            pl.BlockSpec((blk, 8), lambda i: (i, 0)),
        ],
        out_shape=[
            jax.ShapeDtypeStruct((n_tc, 1), jnp.int32),
            jax.ShapeDtypeStruct((n_tc, 8), jnp.float32),
        ],
    )(points[:n_tc], gtT, extT)

    cls = jnp.concatenate([cls_t[:, 0], cls_s])
    box = jnp.concatenate([box_t, box_s], axis=0)
    return cls, box
